# Initial kernel scaffold; baseline (speedup 1.0000x reference)
import jax
import jax.numpy as jnp
from jax.experimental import pallas as pl


def kernel(x, table):
    raise NotImplementedError("write your pallas kernel here")


# speedup vs baseline: 1.0658x; 1.0658x over previous
import functools

import jax
import jax.numpy as jnp
from jax import lax
from jax.experimental import pallas as pl
from jax.experimental.pallas import tpu as pltpu
from jax.experimental.pallas import tpu_sc as plsc

VOCAB = 1000
DIM = 1000
BATCH = 16384

_SM_ROWS = 40


def _softmax_body(t_ref, o_ref):
    t = t_ref[...]
    m = jnp.max(t, axis=1, keepdims=True)
    e = jnp.exp(t - m)
    o_ref[...] = e / jnp.sum(e, axis=1, keepdims=True)


def _softmax_table(table):
    return pl.pallas_call(
        _softmax_body,
        grid=(VOCAB // _SM_ROWS,),
        in_specs=[pl.BlockSpec((_SM_ROWS, DIM), lambda i: (i, 0))],
        out_specs=pl.BlockSpec((_SM_ROWS, DIM), lambda i: (i, 0)),
        out_shape=jax.ShapeDtypeStruct((VOCAB, DIM), jnp.float32),
    )(table)


_NC = 2
_NS = 16
_NW = _NC * _NS
_B_PER_W = BATCH // _NW
_CHUNK = 64
_NCHUNK = _B_PER_W // _CHUNK

_sc_mesh = plsc.VectorSubcoreMesh(core_axis_name="c", subcore_axis_name="s")


@functools.partial(
    pl.kernel,
    out_type=jax.ShapeDtypeStruct((BATCH, DIM), jnp.float32),
    mesh=_sc_mesh,
    scratch_types=[
        pltpu.VMEM((_B_PER_W,), jnp.int32),
        pltpu.VMEM((2, _CHUNK, DIM), jnp.float32),
        pltpu.SemaphoreType.DMA,
        pltpu.SemaphoreType.DMA,
    ],
    compiler_params=pltpu.CompilerParams(use_tc_tiling_on_sc=False),
)
def _gather_rows(table_hbm, idx_hbm, out_hbm, idx_v, rows_v, sem0, sem1):
    wid = lax.axis_index("s") * _NC + lax.axis_index("c")
    base = wid * _B_PER_W
    pltpu.sync_copy(idx_hbm.at[pl.ds(base, _B_PER_W)], idx_v)
    sems = (sem0, sem1)

    def start_gather(g):
        return pltpu.async_copy(
            table_hbm.at[idx_v.at[pl.ds(g * _CHUNK, _CHUNK)]],
            rows_v.at[g % 2],
            sems[g % 2],
        )

    copies = {0: start_gather(0), 1: start_gather(1)}
    for g in range(_NCHUNK):
        copies[g].wait()
        pltpu.sync_copy(rows_v.at[g % 2], out_hbm.at[pl.ds(base + g * _CHUNK, _CHUNK)])
        if g + 2 < _NCHUNK:
            copies[g + 2] = start_gather(g + 2)


def kernel(x, table):
    table_sm = _softmax_table(table)
    return _gather_rows(table_sm, x.astype(jnp.int32))


# speedup vs baseline: 1.0953x; 1.0276x over previous
import functools

import jax
import jax.numpy as jnp
from jax import lax
from jax.experimental import pallas as pl
from jax.experimental.pallas import tpu as pltpu
from jax.experimental.pallas import tpu_sc as plsc

VOCAB = 1000
DIM = 1000
DIM_PAD = 1024
BATCH = 16384
_NEG = -1e30

_NC = 2
_NS = 16
_NW = _NC * _NS
_B_PER_W = BATCH // _NW
_CHUNK = 32
_NCHUNK = _B_PER_W // _CHUNK

_sc_mesh = plsc.VectorSubcoreMesh(core_axis_name="c", subcore_axis_name="s")


@functools.partial(
    pl.kernel,
    out_type=jax.ShapeDtypeStruct((BATCH, DIM_PAD), jnp.float32),
    mesh=_sc_mesh,
    scratch_types=[
        pltpu.VMEM((_B_PER_W,), jnp.int32),
        pltpu.VMEM((2, _CHUNK, DIM_PAD), jnp.float32),
        pltpu.SemaphoreType.DMA,
        pltpu.SemaphoreType.DMA,
    ],
)
def _gather_rows(table_hbm, idx_hbm, out_hbm, idx_v, rows_v, sem0, sem1):
    wid = lax.axis_index("s") * _NC + lax.axis_index("c")
    base = wid * _B_PER_W
    pltpu.sync_copy(idx_hbm.at[pl.ds(base, _B_PER_W)], idx_v)
    sems = (sem0, sem1)

    def start_gather(g):
        return pltpu.async_copy(
            table_hbm.at[idx_v.at[pl.ds(g * _CHUNK, _CHUNK)]],
            rows_v.at[g % 2],
            sems[g % 2],
        )

    copies = {0: start_gather(0), 1: start_gather(1)}
    for g in range(_NCHUNK):
        copies[g].wait()
        pltpu.sync_copy(
            rows_v.at[g % 2], out_hbm.at[pl.ds(base + g * _CHUNK, _CHUNK)]
        )
        if g + 2 < _NCHUNK:
            copies[g + 2] = start_gather(g + 2)


_SM_ROWS = 256


def _softmax_body(t_ref, o_ref):
    t = t_ref[...]
    m = jnp.max(t, axis=1, keepdims=True)
    e = jnp.exp(t - m)
    s = jnp.sum(e, axis=1, keepdims=True)
    o_ref[...] = e[:, :DIM] / s


def _softmax_rows(rows_pad):
    return pl.pallas_call(
        _softmax_body,
        grid=(BATCH // _SM_ROWS,),
        in_specs=[pl.BlockSpec((_SM_ROWS, DIM_PAD), lambda i: (i, 0))],
        out_specs=pl.BlockSpec((_SM_ROWS, DIM), lambda i: (i, 0)),
        out_shape=jax.ShapeDtypeStruct((BATCH, DIM), jnp.float32),
    )(rows_pad)


def kernel(x, table):
    table_pad = jnp.pad(table, ((0, 0), (0, DIM_PAD - DIM)), constant_values=_NEG)
    rows_pad = _gather_rows(table_pad, x.astype(jnp.int32))
    return _softmax_rows(rows_pad)


# speedup vs baseline: 1.4743x; 1.3461x over previous
import functools

import jax
import jax.numpy as jnp
from jax import lax
from jax.experimental import pallas as pl
from jax.experimental.pallas import tpu as pltpu
from jax.experimental.pallas import tpu_sc as plsc

VOCAB = 1000
DIM = 1000
DIM_PAD = 1024
BATCH = 16384
_NEG = -1e30

_NC = 2
_NS = 16
_NW = _NC * _NS
_B_PER_W = BATCH // _NW
_CHUNK = 32
_NCHUNK = _B_PER_W // _CHUNK

_sc_mesh = plsc.VectorSubcoreMesh(core_axis_name="c", subcore_axis_name="s")


@functools.partial(
    pl.kernel,
    out_type=jax.ShapeDtypeStruct((BATCH, DIM_PAD), jnp.float32),
    mesh=_sc_mesh,
    scratch_types=[
        pltpu.VMEM((_B_PER_W,), jnp.int32),
        pltpu.VMEM((2, _CHUNK, DIM_PAD), jnp.float32),
        pltpu.SemaphoreType.DMA,
        pltpu.SemaphoreType.DMA,
    ],
)
def _gather_rows(table_hbm, idx_hbm, out_hbm, idx_v, rows_v, sem0, sem1):
    wid = lax.axis_index("s") * _NC + lax.axis_index("c")
    base = wid * _B_PER_W
    pltpu.sync_copy(idx_hbm.at[pl.ds(base, _B_PER_W)], idx_v)
    sems = (sem0, sem1)

    def start_gather(g):
        return pltpu.async_copy(
            table_hbm.at[idx_v.at[pl.ds(g * _CHUNK, _CHUNK)]],
            rows_v.at[g % 2],
            sems[g % 2],
        )

    copies = {0: start_gather(0), 1: start_gather(1)}
    for g in range(_NCHUNK):
        copies[g].wait()
        pltpu.sync_copy(
            rows_v.at[g % 2], out_hbm.at[pl.ds(base + g * _CHUNK, _CHUNK)]
        )
        if g + 2 < _NCHUNK:
            copies[g + 2] = start_gather(g + 2)


_SM_ROWS = 256


def _softmax_body(t_ref, o_ref):
    t = t_ref[...]
    m = jnp.max(t, axis=1, keepdims=True)
    e = jnp.exp(t - m)
    s = jnp.sum(e, axis=1, keepdims=True)
    sm = e / s
    o_ref[...] = jnp.transpose(sm)[:DIM, :]


def _softmax_rows_t(rows_pad):
    return pl.pallas_call(
        _softmax_body,
        grid=(BATCH // _SM_ROWS,),
        in_specs=[pl.BlockSpec((_SM_ROWS, DIM_PAD), lambda i: (i, 0))],
        out_specs=pl.BlockSpec((DIM, _SM_ROWS), lambda i: (0, i)),
        out_shape=jax.ShapeDtypeStruct((DIM, BATCH), jnp.float32),
    )(rows_pad)


def kernel(x, table):
    table_pad = jnp.pad(table, ((0, 0), (0, DIM_PAD - DIM)), constant_values=_NEG)
    rows_pad = _gather_rows(table_pad, x.astype(jnp.int32))
    return jnp.transpose(_softmax_rows_t(rows_pad))


# speedup vs baseline: 1.8348x; 1.2445x over previous
import functools

import jax
import jax.numpy as jnp
from jax import lax
from jax.experimental import pallas as pl
from jax.experimental.pallas import tpu as pltpu
from jax.experimental.pallas import tpu_sc as plsc

VOCAB = 1000
DIM = 1000
DIM_PAD = 1024
BATCH = 16384
_NEG = -1e30

_NC = 2
_NS = 16
_NW = _NC * _NS
_B_PER_W = BATCH // _NW
_CHUNK = 32
_NCHUNK = _B_PER_W // _CHUNK

_sc_mesh = plsc.VectorSubcoreMesh(core_axis_name="c", subcore_axis_name="s")


@functools.partial(
    pl.kernel,
    out_type=jax.ShapeDtypeStruct((BATCH, DIM_PAD), jnp.float32),
    mesh=_sc_mesh,
    scratch_types=[
        pltpu.VMEM((_B_PER_W,), jnp.int32),
        pltpu.VMEM((3, _CHUNK, DIM_PAD), jnp.float32),
        pltpu.SemaphoreType.DMA,
        pltpu.SemaphoreType.DMA,
        pltpu.SemaphoreType.DMA,
    ],
)
def _gather_rows(table_hbm, idx_hbm, out_hbm, idx_v, rows_v, sem0, sem1, sem2):
    wid = lax.axis_index("s") * _NC + lax.axis_index("c")
    base = wid * _B_PER_W
    pltpu.sync_copy(idx_hbm.at[pl.ds(base, _B_PER_W)], idx_v)
    sems = (sem0, sem1, sem2)

    def start_gather(g):
        return pltpu.async_copy(
            table_hbm.at[idx_v.at[pl.ds(g * _CHUNK, _CHUNK)]],
            rows_v.at[g % 3],
            sems[g % 3],
        )

    copies = {g: start_gather(g) for g in range(3)}
    for g in range(_NCHUNK):
        copies[g].wait()
        pltpu.sync_copy(
            rows_v.at[g % 3], out_hbm.at[pl.ds(base + g * _CHUNK, _CHUNK)]
        )
        if g + 3 < _NCHUNK:
            copies[g + 3] = start_gather(g + 3)


_SM_ROWS = 1024


def _softmax_body(t_ref, o_ref):
    t = t_ref[...]
    m = jnp.max(t, axis=1, keepdims=True)
    e = jnp.exp(t - m)
    s = jnp.sum(e, axis=1, keepdims=True)
    sm = e / s
    o_ref[...] = jnp.transpose(sm)[:DIM, :]


def _softmax_rows_t(rows_pad):
    return pl.pallas_call(
        _softmax_body,
        grid=(BATCH // _SM_ROWS,),
        in_specs=[pl.BlockSpec((_SM_ROWS, DIM_PAD), lambda i: (i, 0))],
        out_specs=pl.BlockSpec((DIM, _SM_ROWS), lambda i: (0, i)),
        out_shape=jax.ShapeDtypeStruct((DIM, BATCH), jnp.float32),
    )(rows_pad)


def kernel(x, table):
    table_pad = jnp.pad(table, ((0, 0), (0, DIM_PAD - DIM)), constant_values=_NEG)
    rows_pad = _gather_rows(table_pad, x.astype(jnp.int32))
    return jnp.transpose(_softmax_rows_t(rows_pad))
